# baseline (speedup 1.0000x reference)
import functools

import jax
import jax.numpy as jnp
from jax import lax
from jax.experimental import pallas as pl
from jax.experimental.pallas import tpu as pltpu
from jax.experimental.pallas import tpu_sc as plsc

Q, K, D = 4096, 16384, 1024
TOPK = 8

BQ = 256
BK = 8192
NQ = Q // BQ
NK = K // BK
SEED = 128
W = SEED + BK

_NEG = float("-inf")
_BIG = 3.0e7


def _normalize_body(x_ref, o_ref):
    x = x_ref[...]
    nrm = jnp.sqrt(jnp.sum(x * x, axis=1, keepdims=True))
    o_ref[...] = (x / (nrm + 1e-8)).astype(jnp.bfloat16)


def _normalize_bf16(x, n_rows, block_rows):
    return pl.pallas_call(
        _normalize_body,
        grid=(n_rows // block_rows,),
        in_specs=[pl.BlockSpec((block_rows, D), lambda i: (i, 0))],
        out_specs=pl.BlockSpec((block_rows, D), lambda i: (i, 0)),
        out_shape=jax.ShapeDtypeStruct((n_rows, D), jnp.bfloat16),
        compiler_params=pltpu.CompilerParams(
            dimension_semantics=("arbitrary",)),
    )(x)


def _topk_body(q_ref, m_ref, imp_ref, dec_ref, idx_ref, v_ref, i_ref):
    ki = pl.program_id(0)
    qi = pl.program_id(1)
    rows = pl.ds(qi * BQ, BQ)

    @pl.when(ki == 0)
    def _():
        v_ref[rows, :] = jnp.full((BQ, TOPK), _NEG, jnp.float32)
        i_ref[rows, :] = jnp.full((BQ, TOPK), _BIG, jnp.float32)

    s = lax.dot_general(q_ref[...], m_ref[...],
                        (((1,), (1,)), ((), ())),
                        preferred_element_type=jnp.float32)
    s = s * (imp_ref[...] * dec_ref[...])

    sa0 = jnp.concatenate(
        [v_ref[rows, :], jnp.full((BQ, SEED - TOPK), _NEG, jnp.float32)],
        axis=1)
    ia0 = jnp.concatenate(
        [i_ref[rows, :], jnp.full((BQ, SEED - TOPK), _BIG, jnp.float32)],
        axis=1)

    lane = lax.broadcasted_iota(jnp.int32, (BQ, SEED), 1).astype(jnp.float32)
    v1 = jnp.full((BQ, SEED), _NEG, jnp.float32)
    v2 = v1
    v3 = v1
    i1 = jnp.full((BQ, SEED), _BIG, jnp.float32)
    i2 = i1
    i3 = i1
    for c in range(BK // SEED):
        x = s[:, c * SEED:(c + 1) * SEED]
        gc = lane + jnp.float32(ki * BK + c * SEED)
        gt1 = x > v1
        gt2 = x > v2
        gt3 = x > v3
        v3 = jnp.where(gt2, v2, jnp.where(gt3, x, v3))
        i3 = jnp.where(gt2, i2, jnp.where(gt3, gc, i3))
        v2 = jnp.where(gt1, v1, jnp.where(gt2, x, v2))
        i2 = jnp.where(gt1, i1, jnp.where(gt2, gc, i2))
        v1 = jnp.where(gt1, x, v1)
        i1 = jnp.where(gt1, gc, i1)

    sa = sa0
    ia = ia0
    cnt = jnp.zeros((BQ, SEED), jnp.float32)
    nv, ni = [], []
    for _ in range(TOPK):
        mx = jnp.maximum(jnp.max(v1, axis=1, keepdims=True),
                         jnp.max(sa, axis=1, keepdims=True))
        am = jnp.minimum(
            jnp.min(jnp.where(v1 == mx, i1, _BIG), axis=1, keepdims=True),
            jnp.min(jnp.where(sa == mx, ia, _BIG), axis=1, keepdims=True))
        nv.append(mx)
        ni.append(am)
        winl = i1 == am
        sa = jnp.where(ia == am, _NEG, sa)
        cnt = cnt + jnp.where(winl, 1.0, 0.0)
        v1 = jnp.where(winl, v2, v1)
        i1 = jnp.where(winl, i2, i1)
        v2 = jnp.where(winl, v3, v2)
        i2 = jnp.where(winl, i3, i2)
        v3 = jnp.where(winl, _NEG, v3)
        i3 = jnp.where(winl, _BIG, i3)
    v_ref[rows, :] = jnp.concatenate(nv, axis=1)
    i_ref[rows, :] = jnp.concatenate(ni, axis=1)

    @pl.when(jnp.max(cnt) >= 3.0)
    def _():
        sb = s
        sc_, ic_ = sa0, ia0
        iota_b = (lax.broadcasted_iota(jnp.int32, (BQ, BK), 1)
                  .astype(jnp.float32) + jnp.float32(ki * BK))
        fv, fi = [], []
        for _ in range(TOPK):
            mx = jnp.maximum(jnp.max(sc_, axis=1, keepdims=True),
                             jnp.max(sb, axis=1, keepdims=True))
            am = jnp.minimum(
                jnp.min(jnp.where(sc_ == mx, ic_, _BIG), axis=1,
                        keepdims=True),
                jnp.min(jnp.where(sb == mx, iota_b, _BIG), axis=1,
                        keepdims=True))
            fv.append(mx)
            fi.append(am)
            sc_ = jnp.where(ic_ == am, _NEG, sc_)
            sb = jnp.where(iota_b == am, _NEG, sb)
        v_ref[rows, :] = jnp.concatenate(fv, axis=1)
        i_ref[rows, :] = jnp.concatenate(fi, axis=1)

    @pl.when(ki == NK - 1)
    def _():
        idx_ref[...] = i_ref[rows, :].astype(jnp.int32)


def _topk_indices(qn_bf, mn_bf, importance, decay):
    return pl.pallas_call(
        _topk_body,
        grid=(NK, NQ),
        in_specs=[
            pl.BlockSpec((BQ, D), lambda ki, qi: (qi, 0)),
            pl.BlockSpec((BK, D), lambda ki, qi: (ki, 0)),
            pl.BlockSpec((1, BK), lambda ki, qi: (0, ki)),
            pl.BlockSpec((1, BK), lambda ki, qi: (0, ki)),
        ],
        out_specs=pl.BlockSpec((BQ, TOPK), lambda ki, qi: (qi, 0)),
        out_shape=jax.ShapeDtypeStruct((Q, TOPK), jnp.int32),
        scratch_shapes=[
            pltpu.VMEM((Q, TOPK), jnp.float32),
            pltpu.VMEM((Q, TOPK), jnp.float32),
        ],
        compiler_params=pltpu.CompilerParams(
            dimension_semantics=("arbitrary", "arbitrary")),
    )(qn_bf, mn_bf, importance.reshape(1, K), decay.reshape(1, K))


_NC, _NS, _L = 2, 16, 16
_NW = _NC * _NS
_QPW = Q // _NW
_CQ = 4
_CROWS = _CQ * TOPK
_NCHUNK = _QPW // _CQ


def _sc_gather_pool_body(bank_hbm, idx_hbm, out_hbm,
                         idx0, idx1, rows0, rows1, pool_v, sem0, sem1):
    wid = lax.axis_index("s") * _NC + lax.axis_index("c")
    idx_v = (idx0, idx1)
    rows_v = (rows0, rows1)
    sems = (sem0, sem1)

    def fire(it, b):
        base = wid * (_QPW * TOPK) + it * _CROWS
        pltpu.sync_copy(idx_hbm.at[pl.ds(base, _CROWS)], idx_v[b])
        pltpu.async_copy(bank_hbm.at[idx_v[b]], rows_v[b], sems[b])

    def drain_accum_store(it, b):
        pltpu.make_async_copy(bank_hbm.at[idx_v[b]], rows_v[b],
                              sems[b]).wait()
        for q in range(_CQ):
            def acc(g, c, q=q, b=b):
                sl = pl.ds(pl.multiple_of(g * _L, _L), _L)
                v = rows_v[b][TOPK * q, sl]
                for r in range(1, TOPK):
                    v = v + rows_v[b][TOPK * q + r, sl]
                pool_v[q, sl] = v
                return c
            lax.fori_loop(0, D // _L, acc, 0)
        qrow = wid * _QPW + it * _CQ
        pltpu.sync_copy(pool_v, out_hbm.at[pl.ds(qrow, _CQ)])

    fire(0, 0)

    def chunk_pair(it2, carry):
        it_a = it2 * 2
        fire(it_a + 1, 1)
        drain_accum_store(it_a, 0)

        @pl.when(it2 < _NCHUNK // 2 - 1)
        def _():
            fire(it_a + 2, 0)
        drain_accum_store(it_a + 1, 1)
        return carry

    lax.fori_loop(0, _NCHUNK // 2, chunk_pair, 0)


@functools.lru_cache(maxsize=1)
def _sc_gather_pool_kernel():
    return pl.kernel(
        _sc_gather_pool_body,
        out_type=jax.ShapeDtypeStruct((Q, D), jnp.float32),
        mesh=plsc.VectorSubcoreMesh(core_axis_name="c", subcore_axis_name="s",
                                    num_cores=_NC, num_subcores=_NS),
        scratch_types=[
            pltpu.VMEM((_CROWS,), jnp.int32),
            pltpu.VMEM((_CROWS,), jnp.int32),
            pltpu.VMEM((_CROWS, D), jnp.float32),
            pltpu.VMEM((_CROWS, D), jnp.float32),
            pltpu.VMEM((_CQ, D), jnp.float32),
            pltpu.SemaphoreType.DMA,
            pltpu.SemaphoreType.DMA,
        ],
    )


def _sc_gather_pool(bank, idx_flat):
    return _sc_gather_pool_kernel()(bank, idx_flat)


BQ2 = 512


def _decode_body(scale_ref, p_ref, w_ref, b_ref, o_ref):
    acc = lax.dot_general(p_ref[...], w_ref[...],
                          (((1,), (1,)), ((), ())),
                          preferred_element_type=jnp.float32)
    o_ref[...] = acc * scale_ref[0, 0] + b_ref[...]


def _decode(pooled, W_dec, b_dec, scale):
    return pl.pallas_call(
        _decode_body,
        grid=(Q // BQ2,),
        in_specs=[
            pl.BlockSpec(memory_space=pltpu.SMEM),
            pl.BlockSpec((BQ2, D), lambda i: (i, 0)),
            pl.BlockSpec((D, D), lambda i: (0, 0)),
            pl.BlockSpec((1, D), lambda i: (0, 0)),
        ],
        out_specs=pl.BlockSpec((BQ2, D), lambda i: (i, 0)),
        out_shape=jax.ShapeDtypeStruct((Q, D), jnp.float32),
        compiler_params=pltpu.CompilerParams(
            dimension_semantics=("parallel",)),
    )(scale, pooled, W_dec, b_dec.reshape(1, D))


def kernel(query, memory_bank, importance, decay, W_dec, b_dec, top_k):
    qn_bf = _normalize_bf16(query, Q, 512)
    mn_bf = _normalize_bf16(memory_bank, K, 1024)
    idx = _topk_indices(qn_bf, mn_bf, importance, decay)
    pooled = _sc_gather_pool(memory_bank, idx.reshape(Q * TOPK))
    scale = (jnp.float32(1.0) / top_k).astype(jnp.float32).reshape(1, 1)
    return _decode(pooled, W_dec, b_dec, scale)


# speedup vs baseline: 1.0538x; 1.0538x over previous
import functools

import jax
import jax.numpy as jnp
from jax import lax
from jax.experimental import pallas as pl
from jax.experimental.pallas import tpu as pltpu
from jax.experimental.pallas import tpu_sc as plsc

Q, K, D = 4096, 16384, 1024
TOPK = 8

BQ = 256
BK = 4096
NQ = Q // BQ
NK = K // BK
SEED = 128
W = SEED + BK

_NEG = float("-inf")
_BIG = 3.0e7


def _normalize_body(x_ref, o_ref):
    x = x_ref[...]
    nrm = jnp.sqrt(jnp.sum(x * x, axis=1, keepdims=True))
    o_ref[...] = (x / (nrm + 1e-8)).astype(jnp.bfloat16)


def _normalize_bf16(x, n_rows, block_rows):
    return pl.pallas_call(
        _normalize_body,
        grid=(n_rows // block_rows,),
        in_specs=[pl.BlockSpec((block_rows, D), lambda i: (i, 0))],
        out_specs=pl.BlockSpec((block_rows, D), lambda i: (i, 0)),
        out_shape=jax.ShapeDtypeStruct((n_rows, D), jnp.bfloat16),
        compiler_params=pltpu.CompilerParams(
            dimension_semantics=("arbitrary",)),
    )(x)


def _topk_body(q_ref, m_ref, imp_ref, dec_ref, idx_ref, v_ref, i_ref):
    ki = pl.program_id(0)
    qi = pl.program_id(1)
    rows = pl.ds(qi * BQ, BQ)

    @pl.when(ki == 0)
    def _():
        v_ref[rows, :] = jnp.full((BQ, TOPK), _NEG, jnp.float32)
        i_ref[rows, :] = jnp.full((BQ, TOPK), _BIG, jnp.float32)

    s = lax.dot_general(q_ref[...], m_ref[...],
                        (((1,), (1,)), ((), ())),
                        preferred_element_type=jnp.float32)
    s = s * (imp_ref[...] * dec_ref[...])

    sa0 = jnp.concatenate(
        [v_ref[rows, :], jnp.full((BQ, SEED - TOPK), _NEG, jnp.float32)],
        axis=1)
    ia0 = jnp.concatenate(
        [i_ref[rows, :], jnp.full((BQ, SEED - TOPK), _BIG, jnp.float32)],
        axis=1)

    lane = lax.broadcasted_iota(jnp.int32, (BQ, SEED), 1).astype(jnp.float32)
    v1 = jnp.full((BQ, SEED), _NEG, jnp.float32)
    v2 = v1
    v3 = v1
    i1 = jnp.full((BQ, SEED), _BIG, jnp.float32)
    i2 = i1
    i3 = i1
    for c in range(BK // SEED):
        x = s[:, c * SEED:(c + 1) * SEED]
        gc = lane + jnp.float32(ki * BK + c * SEED)
        gt1 = x > v1
        gt2 = x > v2
        gt3 = x > v3
        v3 = jnp.where(gt2, v2, jnp.where(gt3, x, v3))
        i3 = jnp.where(gt2, i2, jnp.where(gt3, gc, i3))
        v2 = jnp.where(gt1, v1, jnp.where(gt2, x, v2))
        i2 = jnp.where(gt1, i1, jnp.where(gt2, gc, i2))
        v1 = jnp.where(gt1, x, v1)
        i1 = jnp.where(gt1, gc, i1)

    sa = sa0
    ia = ia0
    cnt = jnp.zeros((BQ, SEED), jnp.float32)
    nv, ni = [], []
    for _ in range(TOPK):
        mx = jnp.maximum(jnp.max(v1, axis=1, keepdims=True),
                         jnp.max(sa, axis=1, keepdims=True))
        am = jnp.minimum(
            jnp.min(jnp.where(v1 == mx, i1, _BIG), axis=1, keepdims=True),
            jnp.min(jnp.where(sa == mx, ia, _BIG), axis=1, keepdims=True))
        nv.append(mx)
        ni.append(am)
        winl = i1 == am
        sa = jnp.where(ia == am, _NEG, sa)
        cnt = cnt + jnp.where(winl, 1.0, 0.0)
        v1 = jnp.where(winl, v2, v1)
        i1 = jnp.where(winl, i2, i1)
        v2 = jnp.where(winl, v3, v2)
        i2 = jnp.where(winl, i3, i2)
        v3 = jnp.where(winl, _NEG, v3)
        i3 = jnp.where(winl, _BIG, i3)
    v_ref[rows, :] = jnp.concatenate(nv, axis=1)
    i_ref[rows, :] = jnp.concatenate(ni, axis=1)

    @pl.when(jnp.max(cnt) >= 3.0)
    def _():
        sb = s
        sc_, ic_ = sa0, ia0
        iota_b = (lax.broadcasted_iota(jnp.int32, (BQ, BK), 1)
                  .astype(jnp.float32) + jnp.float32(ki * BK))
        fv, fi = [], []
        for _ in range(TOPK):
            mx = jnp.maximum(jnp.max(sc_, axis=1, keepdims=True),
                             jnp.max(sb, axis=1, keepdims=True))
            am = jnp.minimum(
                jnp.min(jnp.where(sc_ == mx, ic_, _BIG), axis=1,
                        keepdims=True),
                jnp.min(jnp.where(sb == mx, iota_b, _BIG), axis=1,
                        keepdims=True))
            fv.append(mx)
            fi.append(am)
            sc_ = jnp.where(ic_ == am, _NEG, sc_)
            sb = jnp.where(iota_b == am, _NEG, sb)
        v_ref[rows, :] = jnp.concatenate(fv, axis=1)
        i_ref[rows, :] = jnp.concatenate(fi, axis=1)

    @pl.when(ki == NK - 1)
    def _():
        idx_ref[...] = i_ref[rows, :].astype(jnp.int32)


def _topk_indices(qn_bf, mn_bf, importance, decay):
    return pl.pallas_call(
        _topk_body,
        grid=(NK, NQ),
        in_specs=[
            pl.BlockSpec((BQ, D), lambda ki, qi: (qi, 0)),
            pl.BlockSpec((BK, D), lambda ki, qi: (ki, 0)),
            pl.BlockSpec((1, BK), lambda ki, qi: (0, ki)),
            pl.BlockSpec((1, BK), lambda ki, qi: (0, ki)),
        ],
        out_specs=pl.BlockSpec((BQ, TOPK), lambda ki, qi: (qi, 0)),
        out_shape=jax.ShapeDtypeStruct((Q, TOPK), jnp.int32),
        scratch_shapes=[
            pltpu.VMEM((Q, TOPK), jnp.float32),
            pltpu.VMEM((Q, TOPK), jnp.float32),
        ],
        compiler_params=pltpu.CompilerParams(
            dimension_semantics=("arbitrary", "arbitrary")),
    )(qn_bf, mn_bf, importance.reshape(1, K), decay.reshape(1, K))


_NC, _NS, _L = 2, 16, 16
_NW = _NC * _NS
_QPW = Q // _NW
_CQ = 4
_CROWS = _CQ * TOPK
_NCHUNK = _QPW // _CQ


def _sc_gather_pool_body(bank_hbm, idx_hbm, out_hbm,
                         idx0, idx1, rows0, rows1, pool_v, sem0, sem1):
    wid = lax.axis_index("s") * _NC + lax.axis_index("c")
    idx_v = (idx0, idx1)
    rows_v = (rows0, rows1)
    sems = (sem0, sem1)

    def fire(it, b):
        base = wid * (_QPW * TOPK) + it * _CROWS
        pltpu.sync_copy(idx_hbm.at[pl.ds(base, _CROWS)], idx_v[b])
        pltpu.async_copy(bank_hbm.at[idx_v[b]], rows_v[b], sems[b])

    def drain_accum_store(it, b):
        pltpu.make_async_copy(bank_hbm.at[idx_v[b]], rows_v[b],
                              sems[b]).wait()
        for q in range(_CQ):
            def acc(g, c, q=q, b=b):
                sl = pl.ds(pl.multiple_of(g * _L, _L), _L)
                v = rows_v[b][TOPK * q, sl]
                for r in range(1, TOPK):
                    v = v + rows_v[b][TOPK * q + r, sl]
                pool_v[q, sl] = v
                return c
            lax.fori_loop(0, D // _L, acc, 0)
        qrow = wid * _QPW + it * _CQ
        pltpu.sync_copy(pool_v, out_hbm.at[pl.ds(qrow, _CQ)])

    fire(0, 0)

    def chunk_pair(it2, carry):
        it_a = it2 * 2
        fire(it_a + 1, 1)
        drain_accum_store(it_a, 0)

        @pl.when(it2 < _NCHUNK // 2 - 1)
        def _():
            fire(it_a + 2, 0)
        drain_accum_store(it_a + 1, 1)
        return carry

    lax.fori_loop(0, _NCHUNK // 2, chunk_pair, 0)


@functools.lru_cache(maxsize=1)
def _sc_gather_pool_kernel():
    return pl.kernel(
        _sc_gather_pool_body,
        out_type=jax.ShapeDtypeStruct((Q, D), jnp.float32),
        mesh=plsc.VectorSubcoreMesh(core_axis_name="c", subcore_axis_name="s",
                                    num_cores=_NC, num_subcores=_NS),
        scratch_types=[
            pltpu.VMEM((_CROWS,), jnp.int32),
            pltpu.VMEM((_CROWS,), jnp.int32),
            pltpu.VMEM((_CROWS, D), jnp.float32),
            pltpu.VMEM((_CROWS, D), jnp.float32),
            pltpu.VMEM((_CQ, D), jnp.float32),
            pltpu.SemaphoreType.DMA,
            pltpu.SemaphoreType.DMA,
        ],
    )


def _sc_gather_pool(bank, idx_flat):
    return _sc_gather_pool_kernel()(bank, idx_flat)


BQ2 = 512


def _decode_body(scale_ref, p_ref, w_ref, b_ref, o_ref):
    acc = lax.dot_general(p_ref[...], w_ref[...],
                          (((1,), (1,)), ((), ())),
                          preferred_element_type=jnp.float32)
    o_ref[...] = acc * scale_ref[0, 0] + b_ref[...]


def _decode(pooled, W_dec, b_dec, scale):
    return pl.pallas_call(
        _decode_body,
        grid=(Q // BQ2,),
        in_specs=[
            pl.BlockSpec(memory_space=pltpu.SMEM),
            pl.BlockSpec((BQ2, D), lambda i: (i, 0)),
            pl.BlockSpec((D, D), lambda i: (0, 0)),
            pl.BlockSpec((1, D), lambda i: (0, 0)),
        ],
        out_specs=pl.BlockSpec((BQ2, D), lambda i: (i, 0)),
        out_shape=jax.ShapeDtypeStruct((Q, D), jnp.float32),
        compiler_params=pltpu.CompilerParams(
            dimension_semantics=("parallel",)),
    )(scale, pooled, W_dec, b_dec.reshape(1, D))


def kernel(query, memory_bank, importance, decay, W_dec, b_dec, top_k):
    qn_bf = _normalize_bf16(query, Q, 512)
    mn_bf = _normalize_bf16(memory_bank, K, 1024)
    idx = _topk_indices(qn_bf, mn_bf, importance, decay)
    pooled = _sc_gather_pool(memory_bank, idx.reshape(Q * TOPK))
    scale = (jnp.float32(1.0) / top_k).astype(jnp.float32).reshape(1, 1)
    return _decode(pooled, W_dec, b_dec, scale)


# speedup vs baseline: 1.0731x; 1.0183x over previous
import functools

import jax
import jax.numpy as jnp
from jax import lax
from jax.experimental import pallas as pl
from jax.experimental.pallas import tpu as pltpu
from jax.experimental.pallas import tpu_sc as plsc

Q, K, D = 4096, 16384, 1024
TOPK = 8

BQ = 256
BK = 4096
NQ = Q // BQ
NK = K // BK
SEED = 128
W = SEED + BK

_NEG = float("-inf")
_BIG = 3.0e7


def _normalize_body(x_ref, o_ref):
    x = x_ref[...]
    nrm = jnp.sqrt(jnp.sum(x * x, axis=1, keepdims=True))
    o_ref[...] = (x / (nrm + 1e-8)).astype(jnp.bfloat16)


def _normalize_bf16(x, n_rows, block_rows):
    return pl.pallas_call(
        _normalize_body,
        grid=(n_rows // block_rows,),
        in_specs=[pl.BlockSpec((block_rows, D), lambda i: (i, 0))],
        out_specs=pl.BlockSpec((block_rows, D), lambda i: (i, 0)),
        out_shape=jax.ShapeDtypeStruct((n_rows, D), jnp.bfloat16),
        compiler_params=pltpu.CompilerParams(
            dimension_semantics=("arbitrary",)),
    )(x)


def _topk_body(q_ref, m_ref, imp_ref, dec_ref, idx_ref, v_ref, i_ref):
    ki = pl.program_id(0)
    qi = pl.program_id(1)
    rows = pl.ds(qi * BQ, BQ)

    @pl.when(ki == 0)
    def _():
        v_ref[rows, :] = jnp.full((BQ, TOPK), _NEG, jnp.float32)
        i_ref[rows, :] = jnp.full((BQ, TOPK), _BIG, jnp.float32)

    s = lax.dot_general(q_ref[...], m_ref[...],
                        (((1,), (1,)), ((), ())),
                        preferred_element_type=jnp.float32)
    s = s * (imp_ref[...] * dec_ref[...])

    sa0 = jnp.concatenate(
        [v_ref[rows, :], jnp.full((BQ, SEED - TOPK), _NEG, jnp.float32)],
        axis=1)
    ia0 = jnp.concatenate(
        [i_ref[rows, :], jnp.full((BQ, SEED - TOPK), _BIG, jnp.float32)],
        axis=1)

    lane = lax.broadcasted_iota(jnp.int32, (BQ, SEED), 1).astype(jnp.float32)
    v1 = jnp.full((BQ, SEED), _NEG, jnp.float32)
    v2 = v1
    v3 = v1
    i1 = jnp.full((BQ, SEED), _BIG, jnp.float32)
    i2 = i1
    i3 = i1
    for c in range(BK // SEED):
        x = s[:, c * SEED:(c + 1) * SEED]
        gc = lane + jnp.float32(ki * BK + c * SEED)
        gt1 = x > v1
        gt2 = x > v2
        gt3 = x > v3
        v3 = jnp.where(gt2, v2, jnp.where(gt3, x, v3))
        i3 = jnp.where(gt2, i2, jnp.where(gt3, gc, i3))
        v2 = jnp.where(gt1, v1, jnp.where(gt2, x, v2))
        i2 = jnp.where(gt1, i1, jnp.where(gt2, gc, i2))
        v1 = jnp.where(gt1, x, v1)
        i1 = jnp.where(gt1, gc, i1)

    sa = sa0
    ia = ia0
    cnt = jnp.zeros((BQ, SEED), jnp.float32)
    nv, ni = [], []
    for _ in range(TOPK):
        mx = jnp.maximum(jnp.max(v1, axis=1, keepdims=True),
                         jnp.max(sa, axis=1, keepdims=True))
        am = jnp.minimum(
            jnp.min(jnp.where(v1 == mx, i1, _BIG), axis=1, keepdims=True),
            jnp.min(jnp.where(sa == mx, ia, _BIG), axis=1, keepdims=True))
        nv.append(mx)
        ni.append(am)
        winl = i1 == am
        sa = jnp.where(ia == am, _NEG, sa)
        cnt = cnt + jnp.where(winl, 1.0, 0.0)
        v1 = jnp.where(winl, v2, v1)
        i1 = jnp.where(winl, i2, i1)
        v2 = jnp.where(winl, v3, v2)
        i2 = jnp.where(winl, i3, i2)
        v3 = jnp.where(winl, _NEG, v3)
        i3 = jnp.where(winl, _BIG, i3)
    v_ref[rows, :] = jnp.concatenate(nv, axis=1)
    i_ref[rows, :] = jnp.concatenate(ni, axis=1)

    @pl.when(jnp.max(cnt) >= 3.0)
    def _():
        sb = s
        sc_, ic_ = sa0, ia0
        iota_b = (lax.broadcasted_iota(jnp.int32, (BQ, BK), 1)
                  .astype(jnp.float32) + jnp.float32(ki * BK))
        fv, fi = [], []
        for _ in range(TOPK):
            mx = jnp.maximum(jnp.max(sc_, axis=1, keepdims=True),
                             jnp.max(sb, axis=1, keepdims=True))
            am = jnp.minimum(
                jnp.min(jnp.where(sc_ == mx, ic_, _BIG), axis=1,
                        keepdims=True),
                jnp.min(jnp.where(sb == mx, iota_b, _BIG), axis=1,
                        keepdims=True))
            fv.append(mx)
            fi.append(am)
            sc_ = jnp.where(ic_ == am, _NEG, sc_)
            sb = jnp.where(iota_b == am, _NEG, sb)
        v_ref[rows, :] = jnp.concatenate(fv, axis=1)
        i_ref[rows, :] = jnp.concatenate(fi, axis=1)

    @pl.when(ki == NK - 1)
    def _():
        idx_ref[...] = i_ref[rows, :].astype(jnp.int32)


def _topk_indices(qn_bf, mn_bf, importance, decay):
    return pl.pallas_call(
        _topk_body,
        grid=(NK, NQ),
        in_specs=[
            pl.BlockSpec((BQ, D), lambda ki, qi: (qi, 0)),
            pl.BlockSpec((BK, D), lambda ki, qi: (ki, 0)),
            pl.BlockSpec((1, BK), lambda ki, qi: (0, ki)),
            pl.BlockSpec((1, BK), lambda ki, qi: (0, ki)),
        ],
        out_specs=pl.BlockSpec((BQ, TOPK), lambda ki, qi: (qi, 0)),
        out_shape=jax.ShapeDtypeStruct((Q, TOPK), jnp.int32),
        scratch_shapes=[
            pltpu.VMEM((Q, TOPK), jnp.float32),
            pltpu.VMEM((Q, TOPK), jnp.float32),
        ],
        compiler_params=pltpu.CompilerParams(
            dimension_semantics=("arbitrary", "arbitrary")),
    )(qn_bf, mn_bf, importance.reshape(1, K), decay.reshape(1, K))


_NC, _NS, _L = 2, 16, 16
_NW = _NC * _NS
_QPW = Q // _NW
_CQ = 4
_CROWS = _CQ * TOPK
_NCHUNK = _QPW // _CQ


def _sc_gather_pool_body(bank_hbm, idx_hbm, out_hbm,
                         idx_all, rows0, rows1, pool_v, sem0, sem1):
    wid = lax.axis_index("s") * _NC + lax.axis_index("c")
    rows_v = (rows0, rows1)
    sems = (sem0, sem1)

    pltpu.sync_copy(idx_hbm.at[pl.ds(wid * (_QPW * TOPK), _QPW * TOPK)],
                    idx_all)

    def fire(it, b):
        pltpu.async_copy(bank_hbm.at[idx_all.at[pl.ds(it * _CROWS, _CROWS)]],
                         rows_v[b], sems[b])

    def drain_accum_store(it, b):
        pltpu.make_async_copy(
            bank_hbm.at[idx_all.at[pl.ds(it * _CROWS, _CROWS)]],
            rows_v[b], sems[b]).wait()
        for q in range(_CQ):
            def acc(g, c, q=q, b=b):
                sl = pl.ds(pl.multiple_of(g * _L, _L), _L)
                v = rows_v[b][TOPK * q, sl]
                for r in range(1, TOPK):
                    v = v + rows_v[b][TOPK * q + r, sl]
                pool_v[q, sl] = v
                return c
            lax.fori_loop(0, D // _L, acc, 0)
        qrow = wid * _QPW + it * _CQ
        pltpu.sync_copy(pool_v, out_hbm.at[pl.ds(qrow, _CQ)])

    fire(0, 0)

    def chunk_pair(it2, carry):
        it_a = it2 * 2
        fire(it_a + 1, 1)
        drain_accum_store(it_a, 0)

        @pl.when(it2 < _NCHUNK // 2 - 1)
        def _():
            fire(it_a + 2, 0)
        drain_accum_store(it_a + 1, 1)
        return carry

    lax.fori_loop(0, _NCHUNK // 2, chunk_pair, 0)


@functools.lru_cache(maxsize=1)
def _sc_gather_pool_kernel():
    return pl.kernel(
        _sc_gather_pool_body,
        out_type=jax.ShapeDtypeStruct((Q, D), jnp.float32),
        mesh=plsc.VectorSubcoreMesh(core_axis_name="c", subcore_axis_name="s",
                                    num_cores=_NC, num_subcores=_NS),
        scratch_types=[
            pltpu.VMEM((_QPW * TOPK,), jnp.int32),
            pltpu.VMEM((_CROWS, D), jnp.float32),
            pltpu.VMEM((_CROWS, D), jnp.float32),
            pltpu.VMEM((_CQ, D), jnp.float32),
            pltpu.SemaphoreType.DMA,
            pltpu.SemaphoreType.DMA,
        ],
    )


def _sc_gather_pool(bank, idx_flat):
    return _sc_gather_pool_kernel()(bank, idx_flat)


BQ2 = 512


def _decode_body(scale_ref, p_ref, w_ref, b_ref, o_ref):
    acc = lax.dot_general(p_ref[...], w_ref[...],
                          (((1,), (1,)), ((), ())),
                          preferred_element_type=jnp.float32)
    o_ref[...] = acc * scale_ref[0, 0] + b_ref[...]


def _decode(pooled, W_dec, b_dec, scale):
    return pl.pallas_call(
        _decode_body,
        grid=(Q // BQ2,),
        in_specs=[
            pl.BlockSpec(memory_space=pltpu.SMEM),
            pl.BlockSpec((BQ2, D), lambda i: (i, 0)),
            pl.BlockSpec((D, D), lambda i: (0, 0)),
            pl.BlockSpec((1, D), lambda i: (0, 0)),
        ],
        out_specs=pl.BlockSpec((BQ2, D), lambda i: (i, 0)),
        out_shape=jax.ShapeDtypeStruct((Q, D), jnp.float32),
        compiler_params=pltpu.CompilerParams(
            dimension_semantics=("parallel",)),
    )(scale, pooled, W_dec, b_dec.reshape(1, D))


def kernel(query, memory_bank, importance, decay, W_dec, b_dec, top_k):
    qn_bf = _normalize_bf16(query, Q, 512)
    mn_bf = _normalize_bf16(memory_bank, K, 1024)
    idx = _topk_indices(qn_bf, mn_bf, importance, decay)
    pooled = _sc_gather_pool(memory_bank, idx.reshape(Q * TOPK))
    scale = (jnp.float32(1.0) / top_k).astype(jnp.float32).reshape(1, 1)
    return _decode(pooled, W_dec, b_dec, scale)


# speedup vs baseline: 1.0862x; 1.0122x over previous
import functools

import jax
import jax.numpy as jnp
from jax import lax
from jax.experimental import pallas as pl
from jax.experimental.pallas import tpu as pltpu
from jax.experimental.pallas import tpu_sc as plsc

Q, K, D = 4096, 16384, 1024
TOPK = 8

BQ = 256
BK = 4096
NQ = Q // BQ
NK = K // BK
SEED = 128
W = SEED + BK

_NEG = float("-inf")
_BIG = 3.0e7


def _normalize_body(x_ref, o_ref):
    x = x_ref[...]
    nrm = jnp.sqrt(jnp.sum(x * x, axis=1, keepdims=True))
    o_ref[...] = (x / (nrm + 1e-8)).astype(jnp.bfloat16)


def _normalize_bf16(x, n_rows, block_rows):
    return pl.pallas_call(
        _normalize_body,
        grid=(n_rows // block_rows,),
        in_specs=[pl.BlockSpec((block_rows, D), lambda i: (i, 0))],
        out_specs=pl.BlockSpec((block_rows, D), lambda i: (i, 0)),
        out_shape=jax.ShapeDtypeStruct((n_rows, D), jnp.bfloat16),
        compiler_params=pltpu.CompilerParams(
            dimension_semantics=("arbitrary",)),
    )(x)


def _topk_body(q_ref, m_ref, imp_ref, dec_ref, idx_ref, v_ref, i_ref):
    ki = pl.program_id(0)
    qi = pl.program_id(1)
    rows = pl.ds(qi * BQ, BQ)

    @pl.when(ki == 0)
    def _():
        v_ref[rows, :] = jnp.full((BQ, TOPK), _NEG, jnp.float32)
        i_ref[rows, :] = jnp.full((BQ, TOPK), _BIG, jnp.float32)

    s = lax.dot_general(q_ref[...], m_ref[...],
                        (((1,), (1,)), ((), ())),
                        preferred_element_type=jnp.float32)
    s = s * (imp_ref[...] * dec_ref[...])

    sa0 = jnp.concatenate(
        [v_ref[rows, :], jnp.full((BQ, SEED - TOPK), _NEG, jnp.float32)],
        axis=1)
    ia0 = jnp.concatenate(
        [i_ref[rows, :], jnp.full((BQ, SEED - TOPK), _BIG, jnp.float32)],
        axis=1)

    lane = lax.broadcasted_iota(jnp.int32, (BQ, SEED), 1).astype(jnp.float32)
    v1 = jnp.full((BQ, SEED), _NEG, jnp.float32)
    v2 = v1
    v3 = v1
    i1 = jnp.full((BQ, SEED), _BIG, jnp.float32)
    i2 = i1
    i3 = i1
    for c in range(BK // SEED):
        x = s[:, c * SEED:(c + 1) * SEED]
        gc = lane + jnp.float32(ki * BK + c * SEED)
        gt1 = x > v1
        gt2 = x > v2
        gt3 = x > v3
        v3 = jnp.where(gt2, v2, jnp.where(gt3, x, v3))
        i3 = jnp.where(gt2, i2, jnp.where(gt3, gc, i3))
        v2 = jnp.where(gt1, v1, jnp.where(gt2, x, v2))
        i2 = jnp.where(gt1, i1, jnp.where(gt2, gc, i2))
        v1 = jnp.where(gt1, x, v1)
        i1 = jnp.where(gt1, gc, i1)

    sa = sa0
    ia = ia0
    cnt = jnp.zeros((BQ, SEED), jnp.float32)
    nv, ni = [], []
    for _ in range(TOPK):
        mx = jnp.maximum(jnp.max(v1, axis=1, keepdims=True),
                         jnp.max(sa, axis=1, keepdims=True))
        am = jnp.minimum(
            jnp.min(jnp.where(v1 == mx, i1, _BIG), axis=1, keepdims=True),
            jnp.min(jnp.where(sa == mx, ia, _BIG), axis=1, keepdims=True))
        nv.append(mx)
        ni.append(am)
        winl = i1 == am
        sa = jnp.where(ia == am, _NEG, sa)
        cnt = cnt + jnp.where(winl, 1.0, 0.0)
        v1 = jnp.where(winl, v2, v1)
        i1 = jnp.where(winl, i2, i1)
        v2 = jnp.where(winl, v3, v2)
        i2 = jnp.where(winl, i3, i2)
        v3 = jnp.where(winl, _NEG, v3)
        i3 = jnp.where(winl, _BIG, i3)
    v_ref[rows, :] = jnp.concatenate(nv, axis=1)
    i_ref[rows, :] = jnp.concatenate(ni, axis=1)

    @pl.when(jnp.max(cnt) >= 3.0)
    def _():
        sb = s
        sc_, ic_ = sa0, ia0
        iota_b = (lax.broadcasted_iota(jnp.int32, (BQ, BK), 1)
                  .astype(jnp.float32) + jnp.float32(ki * BK))
        fv, fi = [], []
        for _ in range(TOPK):
            mx = jnp.maximum(jnp.max(sc_, axis=1, keepdims=True),
                             jnp.max(sb, axis=1, keepdims=True))
            am = jnp.minimum(
                jnp.min(jnp.where(sc_ == mx, ic_, _BIG), axis=1,
                        keepdims=True),
                jnp.min(jnp.where(sb == mx, iota_b, _BIG), axis=1,
                        keepdims=True))
            fv.append(mx)
            fi.append(am)
            sc_ = jnp.where(ic_ == am, _NEG, sc_)
            sb = jnp.where(iota_b == am, _NEG, sb)
        v_ref[rows, :] = jnp.concatenate(fv, axis=1)
        i_ref[rows, :] = jnp.concatenate(fi, axis=1)

    @pl.when(ki == NK - 1)
    def _():
        idx_ref[...] = i_ref[rows, :].astype(jnp.int32)


def _topk_indices(qn_bf, mn_bf, importance, decay):
    return pl.pallas_call(
        _topk_body,
        grid=(NK, NQ),
        in_specs=[
            pl.BlockSpec((BQ, D), lambda ki, qi: (qi, 0)),
            pl.BlockSpec((BK, D), lambda ki, qi: (ki, 0)),
            pl.BlockSpec((1, BK), lambda ki, qi: (0, ki)),
            pl.BlockSpec((1, BK), lambda ki, qi: (0, ki)),
        ],
        out_specs=pl.BlockSpec((BQ, TOPK), lambda ki, qi: (qi, 0)),
        out_shape=jax.ShapeDtypeStruct((Q, TOPK), jnp.int32),
        scratch_shapes=[
            pltpu.VMEM((Q, TOPK), jnp.float32),
            pltpu.VMEM((Q, TOPK), jnp.float32),
        ],
        compiler_params=pltpu.CompilerParams(
            dimension_semantics=("arbitrary", "arbitrary")),
    )(qn_bf, mn_bf, importance.reshape(1, K), decay.reshape(1, K))


_NC, _NS, _L = 2, 16, 16
_NW = _NC * _NS
_QPW = Q // _NW
_CQ = 4
_CROWS = _CQ * TOPK
_NCHUNK = _QPW // _CQ


def _sc_gather_pool_body(bank_hbm, idx_hbm, out_hbm,
                         idx_all, rows0, rows1, pool0, pool1,
                         sem0, sem1, st0, st1):
    wid = lax.axis_index("s") * _NC + lax.axis_index("c")
    rows_v = (rows0, rows1)
    pool_v = (pool0, pool1)
    sems = (sem0, sem1)
    ssem = (st0, st1)

    pltpu.sync_copy(idx_hbm.at[pl.ds(wid * (_QPW * TOPK), _QPW * TOPK)],
                    idx_all)

    def fire(it, b):
        pltpu.async_copy(bank_hbm.at[idx_all.at[pl.ds(it * _CROWS, _CROWS)]],
                         rows_v[b], sems[b])

    def drain_accum_store(it, b):
        pltpu.make_async_copy(
            bank_hbm.at[idx_all.at[pl.ds(it * _CROWS, _CROWS)]],
            rows_v[b], sems[b]).wait()

        @pl.when(it >= 2)
        def _():
            pltpu.make_async_copy(pool_v[b], out_hbm.at[pl.ds(0, _CQ)],
                                  ssem[b]).wait()
        for q in range(_CQ):
            def acc(g, c, q=q, b=b):
                sl = pl.ds(pl.multiple_of(g * _L, _L), _L)
                v = rows_v[b][TOPK * q, sl]
                for r in range(1, TOPK):
                    v = v + rows_v[b][TOPK * q + r, sl]
                pool_v[b][q, sl] = v
                return c
            lax.fori_loop(0, D // _L, acc, 0)
        qrow = wid * _QPW + it * _CQ
        pltpu.async_copy(pool_v[b], out_hbm.at[pl.ds(qrow, _CQ)], ssem[b])

    fire(0, 0)

    def chunk_pair(it2, carry):
        it_a = it2 * 2
        fire(it_a + 1, 1)
        drain_accum_store(it_a, 0)

        @pl.when(it2 < _NCHUNK // 2 - 1)
        def _():
            fire(it_a + 2, 0)
        drain_accum_store(it_a + 1, 1)
        return carry

    lax.fori_loop(0, _NCHUNK // 2, chunk_pair, 0)
    pltpu.make_async_copy(pool0, out_hbm.at[pl.ds(0, _CQ)], st0).wait()
    pltpu.make_async_copy(pool1, out_hbm.at[pl.ds(0, _CQ)], st1).wait()


@functools.lru_cache(maxsize=1)
def _sc_gather_pool_kernel():
    return pl.kernel(
        _sc_gather_pool_body,
        out_type=jax.ShapeDtypeStruct((Q, D), jnp.float32),
        mesh=plsc.VectorSubcoreMesh(core_axis_name="c", subcore_axis_name="s",
                                    num_cores=_NC, num_subcores=_NS),
        scratch_types=[
            pltpu.VMEM((_QPW * TOPK,), jnp.int32),
            pltpu.VMEM((_CROWS, D), jnp.float32),
            pltpu.VMEM((_CROWS, D), jnp.float32),
            pltpu.VMEM((_CQ, D), jnp.float32),
            pltpu.VMEM((_CQ, D), jnp.float32),
            pltpu.SemaphoreType.DMA,
            pltpu.SemaphoreType.DMA,
            pltpu.SemaphoreType.DMA,
            pltpu.SemaphoreType.DMA,
        ],
    )


def _sc_gather_pool(bank, idx_flat):
    return _sc_gather_pool_kernel()(bank, idx_flat)


BQ2 = 512


def _decode_body(scale_ref, p_ref, w_ref, b_ref, o_ref):
    acc = lax.dot_general(p_ref[...], w_ref[...],
                          (((1,), (1,)), ((), ())),
                          preferred_element_type=jnp.float32)
    o_ref[...] = acc * scale_ref[0, 0] + b_ref[...]


def _decode(pooled, W_dec, b_dec, scale):
    return pl.pallas_call(
        _decode_body,
        grid=(Q // BQ2,),
        in_specs=[
            pl.BlockSpec(memory_space=pltpu.SMEM),
            pl.BlockSpec((BQ2, D), lambda i: (i, 0)),
            pl.BlockSpec((D, D), lambda i: (0, 0)),
            pl.BlockSpec((1, D), lambda i: (0, 0)),
        ],
        out_specs=pl.BlockSpec((BQ2, D), lambda i: (i, 0)),
        out_shape=jax.ShapeDtypeStruct((Q, D), jnp.float32),
        compiler_params=pltpu.CompilerParams(
            dimension_semantics=("parallel",)),
    )(scale, pooled, W_dec, b_dec.reshape(1, D))


def kernel(query, memory_bank, importance, decay, W_dec, b_dec, top_k):
    qn_bf = _normalize_bf16(query, Q, 512)
    mn_bf = _normalize_bf16(memory_bank, K, 1024)
    idx = _topk_indices(qn_bf, mn_bf, importance, decay)
    pooled = _sc_gather_pool(memory_bank, idx.reshape(Q * TOPK))
    scale = (jnp.float32(1.0) / top_k).astype(jnp.float32).reshape(1, 1)
    return _decode(pooled, W_dec, b_dec, scale)
